# Initial kernel scaffold; baseline (speedup 1.0000x reference)
import jax
import jax.numpy as jnp
from jax.experimental import pallas as pl


def kernel(word_h, heads, types, instances, W_fw0, W_bw0, W_fw1, W_bw1, b_fw0, b_bw0, b_fw1, b_bw1):
    raise NotImplementedError("write your pallas kernel here")


# speedup vs baseline: 20.2290x; 20.2290x over previous
import functools

import jax
import jax.numpy as jnp
from jax import lax
from jax.experimental import pallas as pl
from jax.experimental.pallas import tpu as pltpu

B, L, H, SPAN = 32, 256, 256, 4


def _body(inst_ref, heads_ref, x_ref,
          wf0_ref, wb0_ref, wf1_ref, wb1_ref,
          bf0_ref, bb0_ref, bf1_ref, bb1_ref,
          out_ref):
    b = pl.program_id(0)
    heads = heads_ref[0]
    row_i = lax.broadcasted_iota(jnp.int32, (L, L), 0)
    col_i = lax.broadcasted_iota(jnp.int32, (L, L), 1)
    eye = (row_i == col_i).astype(jnp.float32)
    dep_fw = (heads == row_i).astype(jnp.float32) + eye
    dep_bw = (heads.reshape(L, 1) == col_i).astype(jnp.float32) + eye

    def gcn(x, dep, w_ref, b_ref):
        y = jnp.dot(x, w_ref[...], preferred_element_type=jnp.float32)
        y = y + b_ref[...]
        return jnp.maximum(jnp.dot(dep, y, preferred_element_type=jnp.float32), 0.0)

    x = x_ref[0]
    h1 = jnp.concatenate(
        [gcn(x, dep_fw, wf0_ref, bf0_ref), gcn(x, dep_bw, wb0_ref, bb0_ref)], axis=1)
    h2 = jnp.concatenate(
        [gcn(h1, dep_fw, wf1_ref, bf1_ref), gcn(h1, dep_bw, wb1_ref, bb1_ref)], axis=1)

    s1 = inst_ref[4 * b + 0]
    s2 = inst_ref[4 * b + 2]
    col2 = lax.broadcasted_iota(jnp.int32, (2, L), 1)
    srow = jnp.where(lax.broadcasted_iota(jnp.int32, (2, L), 0) == 0, s1, s2)
    sel = jnp.where((col2 > srow) & (col2 <= srow + SPAN), 1.0 / SPAN, 0.0)
    res = jnp.dot(sel, h2, preferred_element_type=jnp.float32)
    out_ref[0] = res.reshape(1, 4 * H)


@jax.jit
def kernel(word_h, heads, types, instances,
           W_fw0, W_bw0, W_fw1, W_bw1, b_fw0, b_bw0, b_fw1, b_bw1):
    del types
    heads3 = heads.astype(jnp.int32).reshape(B, 1, L)
    inst_flat = instances.astype(jnp.int32).reshape(B * 4)
    biases = [b.reshape(1, H) for b in (b_fw0, b_bw0, b_fw1, b_bw1)]

    grid_spec = pltpu.PrefetchScalarGridSpec(
        num_scalar_prefetch=1,
        grid=(B,),
        in_specs=[
            pl.BlockSpec((1, 1, L), lambda b, inst: (b, 0, 0)),
            pl.BlockSpec((1, L, 2 * H), lambda b, inst: (b, 0, 0)),
            pl.BlockSpec((2 * H, H), lambda b, inst: (0, 0)),
            pl.BlockSpec((2 * H, H), lambda b, inst: (0, 0)),
            pl.BlockSpec((2 * H, H), lambda b, inst: (0, 0)),
            pl.BlockSpec((2 * H, H), lambda b, inst: (0, 0)),
            pl.BlockSpec((1, H), lambda b, inst: (0, 0)),
            pl.BlockSpec((1, H), lambda b, inst: (0, 0)),
            pl.BlockSpec((1, H), lambda b, inst: (0, 0)),
            pl.BlockSpec((1, H), lambda b, inst: (0, 0)),
        ],
        out_specs=pl.BlockSpec((1, 1, 4 * H), lambda b, inst: (b, 0, 0)),
    )
    out = pl.pallas_call(
        _body,
        grid_spec=grid_spec,
        out_shape=jax.ShapeDtypeStruct((B, 1, 4 * H), jnp.float32),
        compiler_params=pltpu.CompilerParams(
            dimension_semantics=("arbitrary",),
        ),
    )(inst_flat, heads3, word_h,
      W_fw0, W_bw0, W_fw1, W_bw1, *biases)
    return out.reshape(B, 4 * H)
